# baseline (speedup 1.0000x reference)
import functools

import jax
import jax.numpy as jnp
from jax import lax
from jax.experimental import pallas as pl
from jax.experimental.pallas import tpu as pltpu
from jax.experimental.pallas import tpu_sc as plsc

F = 8
V = 1000
B = 16384
D = 32
PAIRS = [(i, j) for i in range(F) for j in range(i + 1, F)]
P = len(PAIRS)

NC, NS = 2, 16
NW = NC * NS
BW = B // NW

_f32 = jnp.float32
_i32 = jnp.int32


def _sc_body(feats_hbm, crosses_hbm, lins_hbm, embs_hbm,
             wide_hbm, deep_hbm,
             feats_v, cidx, lidx, cvals, lvals,
             ebuf0, ebuf1, wide_v,
             sem_c, sem_l, sem_d0, sem_d1):
    wid = lax.axis_index("s") * NC + lax.axis_index("c")

    pltpu.sync_copy(feats_hbm.at[:, pl.ds(wid * BW, BW)], feats_v)

    def idx_body(c, _):
        co = c * 16
        fv = [feats_v[f, pl.ds(co, 16)] for f in range(F)]
        for f in range(F):
            lidx[pl.ds(f * BW + co, 16)] = fv[f] + f * V
        for p, (a, b) in enumerate(PAIRS):
            cidx[pl.ds(p * BW + co, 16)] = fv[a] * V + fv[b] + p * (V * V)
        return 0
    lax.fori_loop(0, BW // 16, idx_body, 0)

    cop = pltpu.async_copy(crosses_hbm.at[cidx], cvals, sem_c)
    lop = pltpu.async_copy(lins_hbm.at[lidx], lvals, sem_l)

    bufs = (ebuf0, ebuf1)
    sems = (sem_d0, sem_d1)
    ops = [None, None]
    ops[0] = pltpu.async_copy(embs_hbm.at[lidx.at[pl.ds(0, BW)]],
                              bufs[0], sems[0])
    for f in range(F):
        cur = f % 2
        if f + 1 < F:
            nxt = (f + 1) % 2
            ops[nxt] = pltpu.async_copy(
                embs_hbm.at[lidx.at[pl.ds((f + 1) * BW, BW)]],
                bufs[nxt], sems[nxt])
        ops[cur].wait()
        pltpu.sync_copy(bufs[cur], deep_hbm.at[pl.ds(wid * BW, BW), f, :])

    cop.wait()
    lop.wait()

    def red_body(c, _):
        co = c * 16
        acc = cvals[pl.ds(co, 16)]
        for p in range(1, P):
            acc = acc + cvals[pl.ds(p * BW + co, 16)]
        for f in range(F):
            acc = acc + lvals[pl.ds(f * BW + co, 16)]
        wide_v[pl.ds(co, 16)] = acc
        return 0
    lax.fori_loop(0, BW // 16, red_body, 0)

    pltpu.sync_copy(wide_v, wide_hbm.at[pl.ds(wid * BW, BW)])


@jax.jit
def _sc_gather(feats, crosses_flat, lins_flat, embs_flat):
    mesh = plsc.VectorSubcoreMesh(core_axis_name="c", subcore_axis_name="s",
                                  num_cores=NC, num_subcores=NS)
    return pl.kernel(
        _sc_body,
        out_type=[
            jax.ShapeDtypeStruct((B,), _f32),
            jax.ShapeDtypeStruct((B, F, D), _f32),
        ],
        mesh=mesh,
        compiler_params=pltpu.CompilerParams(use_tc_tiling_on_sc=False),
        scratch_types=[
            pltpu.VMEM((F, BW), _i32),
            pltpu.VMEM((P * BW,), _i32),
            pltpu.VMEM((F * BW,), _i32),
            pltpu.VMEM((P * BW,), _f32),
            pltpu.VMEM((F * BW,), _f32),
            pltpu.VMEM((BW, D), _f32),
            pltpu.VMEM((BW, D), _f32),
            pltpu.VMEM((BW,), _f32),
            pltpu.SemaphoreType.DMA,
            pltpu.SemaphoreType.DMA,
            pltpu.SemaphoreType.DMA,
            pltpu.SemaphoreType.DMA,
        ],
    )(feats, crosses_flat, lins_flat, embs_flat)


def _mlp_body(deep_ref, wide_ref, w1_ref, b1_ref, w2_ref, b2_ref,
              w3_ref, b3_ref, out_ref):
    x = deep_ref[...]
    h = jnp.maximum(jnp.dot(x, w1_ref[...],
                            preferred_element_type=_f32) + b1_ref[...], 0.0)
    h = jnp.maximum(jnp.dot(h, w2_ref[...],
                            preferred_element_type=_f32) + b2_ref[...], 0.0)
    logit = (jnp.dot(h, w3_ref[...], preferred_element_type=_f32)
             + b3_ref[...] + wide_ref[...])
    out_ref[...] = jax.nn.sigmoid(logit)


def _mlp(deep2, wide2, W1, b1, W2, b2, W3, b3):
    bm = 1024
    grid = (B // bm,)
    return pl.pallas_call(
        _mlp_body,
        grid=grid,
        in_specs=[
            pl.BlockSpec((bm, F * D), lambda i: (i, 0)),
            pl.BlockSpec((bm, 1), lambda i: (i, 0)),
            pl.BlockSpec((F * D, 256), lambda i: (0, 0)),
            pl.BlockSpec((1, 256), lambda i: (0, 0)),
            pl.BlockSpec((256, 128), lambda i: (0, 0)),
            pl.BlockSpec((1, 128), lambda i: (0, 0)),
            pl.BlockSpec((128, 1), lambda i: (0, 0)),
            pl.BlockSpec((1, 1), lambda i: (0, 0)),
        ],
        out_specs=pl.BlockSpec((bm, 1), lambda i: (i, 0)),
        out_shape=jax.ShapeDtypeStruct((B, 1), _f32),
    )(deep2, wide2, W1, b1, W2, b2, W3, b3)


def kernel(feats, embs, lins, crosses, W1, b1, W2, b2, W3, b3):
    crosses_flat = crosses.reshape(-1)
    lins_flat = lins.reshape(-1)
    embs_flat = embs.reshape(F * V, D)
    wide, deep = _sc_gather(feats, crosses_flat, lins_flat, embs_flat)
    deep2 = deep.reshape(B, F * D)
    wide2 = wide.reshape(B, 1)
    return _mlp(deep2, wide2, W1, b1.reshape(1, 256), W2, b2.reshape(1, 128),
                W3, b3.reshape(1, 1))


# speedup vs baseline: 1.0189x; 1.0189x over previous
import jax
import jax.numpy as jnp
from jax import lax
from jax.experimental import pallas as pl
from jax.experimental.pallas import tpu as pltpu
from jax.experimental.pallas import tpu_sc as plsc

F = 8
V = 1000
B = 16384
D = 32
PAIRS = [(i, j) for i in range(F) for j in range(i + 1, F)]
P = len(PAIRS)

NC, NS = 2, 16
NW = NC * NS
BW = B // NW

_f32 = jnp.float32
_i32 = jnp.int32


def _sc_body(feats_hbm, crosses_hbm, lins_hbm, embs_hbm,
             wide_hbm, deep_hbm,
             feats_v, cidx, lidx, cvals, lvals,
             ebuf0, ebuf1, wide_v,
             sem_c, sem_l, sem_d0, sem_d1):
    wid = lax.axis_index("s") * NC + lax.axis_index("c")

    pltpu.sync_copy(feats_hbm.at[:, pl.ds(wid * BW, BW)], feats_v)

    def idx_body(c, _):
        co = c * 16
        fv = [feats_v[f, pl.ds(co, 16)] for f in range(F)]
        for f in range(F):
            lidx[pl.ds(f * BW + co, 16)] = fv[f]
        for p, (a, b) in enumerate(PAIRS):
            cidx[pl.ds(p * BW + co, 16)] = fv[a] * V + fv[b]
        return 0
    lax.fori_loop(0, BW // 16, idx_body, 0)

    for p in range(P):
        pltpu.async_copy(
            crosses_hbm.at[p].at[cidx.at[pl.ds(p * BW, BW)]],
            cvals.at[p], sem_c)
    for f in range(F):
        pltpu.async_copy(
            lins_hbm.at[f].at[lidx.at[pl.ds(f * BW, BW)]],
            lvals.at[f], sem_l)

    bufs = (ebuf0, ebuf1)
    sems = (sem_d0, sem_d1)
    ops = [None, None]
    ops[0] = pltpu.async_copy(
        embs_hbm.at[0].at[lidx.at[pl.ds(0, BW)]], bufs[0], sems[0])
    for f in range(F):
        cur = f % 2
        if f + 1 < F:
            nxt = (f + 1) % 2
            ops[nxt] = pltpu.async_copy(
                embs_hbm.at[f + 1].at[lidx.at[pl.ds((f + 1) * BW, BW)]],
                bufs[nxt], sems[nxt])
        ops[cur].wait()
        pltpu.sync_copy(bufs[cur],
                        deep_hbm.at[pl.ds(wid * BW, BW), pl.ds(f * D, D)])

    for p in range(P):
        pltpu.make_async_copy(
            crosses_hbm.at[p].at[cidx.at[pl.ds(p * BW, BW)]],
            cvals.at[p], sem_c).wait()
    for f in range(F):
        pltpu.make_async_copy(
            lins_hbm.at[f].at[lidx.at[pl.ds(f * BW, BW)]],
            lvals.at[f], sem_l).wait()

    def red_body(c, _):
        co = c * 16
        acc = cvals[0, pl.ds(co, 16)]
        for p in range(1, P):
            acc = acc + cvals[p, pl.ds(co, 16)]
        for f in range(F):
            acc = acc + lvals[f, pl.ds(co, 16)]
        wide_v[pl.ds(co, 16)] = acc
        return 0
    lax.fori_loop(0, BW // 16, red_body, 0)

    pltpu.sync_copy(wide_v, wide_hbm.at[pl.ds(wid * BW, BW)])


@jax.jit
def _sc_gather(feats, crosses, lins, embs):
    mesh = plsc.VectorSubcoreMesh(core_axis_name="c", subcore_axis_name="s",
                                  num_cores=NC, num_subcores=NS)
    return pl.kernel(
        _sc_body,
        out_type=[
            jax.ShapeDtypeStruct((B,), _f32),
            jax.ShapeDtypeStruct((B, F * D), _f32),
        ],
        mesh=mesh,
        compiler_params=pltpu.CompilerParams(use_tc_tiling_on_sc=False),
        scratch_types=[
            pltpu.VMEM((F, BW), _i32),
            pltpu.VMEM((P * BW,), _i32),
            pltpu.VMEM((F * BW,), _i32),
            pltpu.VMEM((P, BW), _f32),
            pltpu.VMEM((F, BW), _f32),
            pltpu.VMEM((BW, D), _f32),
            pltpu.VMEM((BW, D), _f32),
            pltpu.VMEM((BW,), _f32),
            pltpu.SemaphoreType.DMA,
            pltpu.SemaphoreType.DMA,
            pltpu.SemaphoreType.DMA,
            pltpu.SemaphoreType.DMA,
        ],
    )(feats, crosses, lins, embs)


def _mlp_body(deep_ref, wide_ref, w1_ref, b1_ref, w2_ref, b2_ref,
              w3_ref, b3_ref, out_ref):
    x = deep_ref[...]
    h = jnp.maximum(jnp.dot(x, w1_ref[...],
                            preferred_element_type=_f32) + b1_ref[...], 0.0)
    h = jnp.maximum(jnp.dot(h, w2_ref[...],
                            preferred_element_type=_f32) + b2_ref[...], 0.0)
    logit = (jnp.dot(h, w3_ref[...], preferred_element_type=_f32)
             + b3_ref[...] + wide_ref[...])
    out_ref[...] = jax.nn.sigmoid(logit)


def _mlp(deep2, wide2, W1, b1, W2, b2, W3, b3):
    bm = 1024
    grid = (B // bm,)
    return pl.pallas_call(
        _mlp_body,
        grid=grid,
        in_specs=[
            pl.BlockSpec((bm, F * D), lambda i: (i, 0)),
            pl.BlockSpec((bm, 1), lambda i: (i, 0)),
            pl.BlockSpec((F * D, 256), lambda i: (0, 0)),
            pl.BlockSpec((1, 256), lambda i: (0, 0)),
            pl.BlockSpec((256, 128), lambda i: (0, 0)),
            pl.BlockSpec((1, 128), lambda i: (0, 0)),
            pl.BlockSpec((128, 1), lambda i: (0, 0)),
            pl.BlockSpec((1, 1), lambda i: (0, 0)),
        ],
        out_specs=pl.BlockSpec((bm, 1), lambda i: (i, 0)),
        out_shape=jax.ShapeDtypeStruct((B, 1), _f32),
    )(deep2, wide2, W1, b1, W2, b2, W3, b3)


def kernel(feats, embs, lins, crosses, W1, b1, W2, b2, W3, b3):
    wide, deep = _sc_gather(feats, crosses.reshape(P, V * V), lins.reshape(F, V), embs)
    wide2 = wide.reshape(B, 1)
    return _mlp(deep, wide2, W1, b1.reshape(1, 256), W2, b2.reshape(1, 128),
                W3, b3.reshape(1, 1))


# speedup vs baseline: 8.0003x; 7.8517x over previous
import jax
import jax.numpy as jnp
from jax import lax
from jax.experimental import pallas as pl
from jax.experimental.pallas import tpu as pltpu
from jax.experimental.pallas import tpu_sc as plsc

F = 8
V = 1000
B = 16384
D = 32
PAIRS = [(i, j) for i in range(F) for j in range(i + 1, F)]
P = len(PAIRS)

NC, NS = 2, 16
NW = NC * NS
BW = B // NW

_f32 = jnp.float32
_i32 = jnp.int32


def _sc_deep_body(feats_hbm, embs_hbm, deep_hbm,
                  feats_v, eidx, ebuf0, ebuf1, sem_d0, sem_d1):
    wid = lax.axis_index("s") * NC + lax.axis_index("c")

    pltpu.sync_copy(feats_hbm.at[:, pl.ds(wid * BW, BW)], feats_v)

    def idx_body(c, _):
        co = c * 16
        for f in range(F):
            eidx[pl.ds(f * BW + co, 16)] = feats_v[f, pl.ds(co, 16)] + f * V
        return 0
    lax.fori_loop(0, BW // 16, idx_body, 0)

    bufs = (ebuf0, ebuf1)
    sems = (sem_d0, sem_d1)
    ops = [None, None]
    ops[0] = pltpu.async_copy(
        embs_hbm.at[eidx.at[pl.ds(0, BW)]], bufs[0], sems[0])
    for f in range(F):
        cur = f % 2
        if f + 1 < F:
            nxt = (f + 1) % 2
            ops[nxt] = pltpu.async_copy(
                embs_hbm.at[eidx.at[pl.ds((f + 1) * BW, BW)]],
                bufs[nxt], sems[nxt])
        ops[cur].wait()
        pltpu.sync_copy(bufs[cur],
                        deep_hbm.at[pl.ds(wid * BW, BW), pl.ds(f * D, D)])


def _sc_wide_body(feats_hbm, crosses_hbm, lins_hbm,
                  wide_hbm,
                  feats_v, cidx, lidx, cvals, lvals, wide_v, sem_a):
    wid = lax.axis_index("s") * NC + lax.axis_index("c")

    pltpu.sync_copy(feats_hbm.at[:, pl.ds(wid * BW, BW)], feats_v)

    def idx_body(c, _):
        co = c * 16
        fv = [feats_v[f, pl.ds(co, 16)] for f in range(F)]
        for f in range(F):
            lv = fv[f]
            lidx[pl.ds(f * BW + co, 16)] = (
                (lv >> 7) * 1024 + f * 128 + (lv & 127))
        for p, (a, b) in enumerate(PAIRS):
            j = fv[a] * V + fv[b]
            base = (p // 8) * (7813 * 1024) + (p % 8) * 128
            cidx[pl.ds(p * BW + co, 16)] = (
                (j >> 7) * 1024 + (j & 127) + base)
        return 0
    lax.fori_loop(0, BW // 16, idx_body, 0)

    cop = pltpu.async_copy(crosses_hbm.at[cidx], cvals, sem_a)
    lop = pltpu.async_copy(lins_hbm.at[lidx], lvals, sem_a)
    cop.wait()
    lop.wait()

    def red_body(c, _):
        co = c * 16
        acc = cvals[pl.ds(co, 16)]
        for p in range(1, P):
            acc = acc + cvals[pl.ds(p * BW + co, 16)]
        for f in range(F):
            acc = acc + lvals[pl.ds(f * BW + co, 16)]
        wide_v[pl.ds(co, 16)] = acc
        return 0
    lax.fori_loop(0, BW // 16, red_body, 0)

    pltpu.sync_copy(wide_v, wide_hbm.at[pl.ds(wid * BW, BW)])


def _sc_mesh():
    return plsc.VectorSubcoreMesh(core_axis_name="c", subcore_axis_name="s",
                                  num_cores=NC, num_subcores=NS)


@jax.jit
def _sc_deep(feats, embs):
    return pl.kernel(
        _sc_deep_body,
        out_type=jax.ShapeDtypeStruct((B, F * D), _f32),
        mesh=_sc_mesh(),
        compiler_params=pltpu.CompilerParams(use_tc_tiling_on_sc=False),
        scratch_types=[
            pltpu.VMEM((F, BW), _i32),
            pltpu.VMEM((F * BW,), _i32),
            pltpu.VMEM((BW, D), _f32),
            pltpu.VMEM((BW, D), _f32),
            pltpu.SemaphoreType.DMA,
            pltpu.SemaphoreType.DMA,
        ],
    )(feats, embs)


@jax.jit
def _sc_wide(feats, crosses, lins):
    return pl.kernel(
        _sc_wide_body,
        out_type=jax.ShapeDtypeStruct((B,), _f32),
        mesh=_sc_mesh(),
        compiler_params=pltpu.CompilerParams(use_tc_tiling_on_sc=False),
        scratch_types=[
            pltpu.VMEM((F, BW), _i32),
            pltpu.VMEM((P * BW,), _i32),
            pltpu.VMEM((F * BW,), _i32),
            pltpu.VMEM((P * BW,), _f32),
            pltpu.VMEM((F * BW,), _f32),
            pltpu.VMEM((BW,), _f32),
            pltpu.SemaphoreType.DMA,
        ],
    )(feats, crosses, lins)


def _mlp_body(deep_ref, w1_ref, b1_ref, w2_ref, b2_ref,
              w3_ref, b3_ref, out_ref):
    x = deep_ref[...]
    h = jnp.maximum(jnp.dot(x, w1_ref[...],
                            preferred_element_type=_f32) + b1_ref[...], 0.0)
    h = jnp.maximum(jnp.dot(h, w2_ref[...],
                            preferred_element_type=_f32) + b2_ref[...], 0.0)
    out_ref[...] = (jnp.dot(h, w3_ref[...], preferred_element_type=_f32)
                    + b3_ref[...])


def _mlp(deep2, W1, b1, W2, b2, W3, b3):
    bm = 2048
    grid = (B // bm,)
    return pl.pallas_call(
        _mlp_body,
        grid=grid,
        in_specs=[
            pl.BlockSpec((bm, F * D), lambda i: (i, 0)),
            pl.BlockSpec((F * D, 256), lambda i: (0, 0)),
            pl.BlockSpec((1, 256), lambda i: (0, 0)),
            pl.BlockSpec((256, 128), lambda i: (0, 0)),
            pl.BlockSpec((1, 128), lambda i: (0, 0)),
            pl.BlockSpec((128, 1), lambda i: (0, 0)),
            pl.BlockSpec((1, 1), lambda i: (0, 0)),
        ],
        out_specs=pl.BlockSpec((bm, 1), lambda i: (i, 0)),
        out_shape=jax.ShapeDtypeStruct((B, 1), _f32),
    )(deep2, W1, b1, W2, b2, W3, b3)


def _final_body(logit_ref, wide_ref, out_ref):
    out_ref[...] = jax.nn.sigmoid(logit_ref[...] + wide_ref[...])


def _final(logit, wide2):
    bm = 8192
    return pl.pallas_call(
        _final_body,
        grid=(B // bm,),
        in_specs=[
            pl.BlockSpec((bm, 1), lambda i: (i, 0)),
            pl.BlockSpec((bm, 1), lambda i: (i, 0)),
        ],
        out_specs=pl.BlockSpec((bm, 1), lambda i: (i, 0)),
        out_shape=jax.ShapeDtypeStruct((B, 1), _f32),
    )(logit, wide2)


def kernel(feats, embs, lins, crosses, W1, b1, W2, b2, W3, b3):
    xp = jnp.pad(crosses.reshape(P, V * V), ((0, 4), (0, 64)))
    crosses_in = jnp.transpose(
        xp.reshape(4, 8, 7813, 128), (0, 2, 1, 3)).reshape(-1)
    lp = jnp.pad(lins.reshape(F, V), ((0, 0), (0, 24)))
    lins_in = jnp.transpose(
        lp.reshape(8, 8, 128), (1, 0, 2)).reshape(-1)
    deep = _sc_deep(feats, embs.reshape(F * V, D))
    wide = _sc_wide(feats, crosses_in, lins_in)
    logit = _mlp(deep, W1, b1.reshape(1, 256), W2, b2.reshape(1, 128),
                 W3, b3.reshape(1, 1))
    return _final(logit, wide.reshape(B, 1))
